# baseline (speedup 1.0000x reference)
import functools

import jax
import jax.numpy as jnp
from jax import lax
from jax.experimental import pallas as pl
from jax.experimental.pallas import tpu as pltpu
from jax.experimental.pallas import tpu_sc as plsc

N_NODES = 10000
N_EDGES = 320000
D_NODE = 128
D_EDGE = 16
D_GLOBAL = 32

NC = 2
NS = 16
NW = NC * NS
EPT = N_EDGES // NW
C = 80
CHUNKS = EPT // C
N_AGG = 10240
ZROWS = N_AGG // NS


def _tc_nodes_body(x_ref, wr_ref, wc_ref, wn_ref, a_ref, b_ref, p_ref):
    x = x_ref[...]
    a_ref[...] = jnp.dot(x, wr_ref[...], preferred_element_type=jnp.float32)
    b_ref[...] = jnp.dot(x, wc_ref[...], preferred_element_type=jnp.float32)
    p_ref[...] = jnp.dot(x, wn_ref[...], preferred_element_type=jnp.float32)


def _tc_nodes(node_attr, We_r, We_c, Wn_n):
    blk = 1000
    grid = (N_NODES // blk,)
    return pl.pallas_call(
        _tc_nodes_body,
        grid=grid,
        in_specs=[
            pl.BlockSpec((blk, D_NODE), lambda i: (i, 0)),
            pl.BlockSpec((D_NODE, D_EDGE), lambda i: (0, 0)),
            pl.BlockSpec((D_NODE, D_EDGE), lambda i: (0, 0)),
            pl.BlockSpec((D_NODE, D_NODE), lambda i: (0, 0)),
        ],
        out_specs=[
            pl.BlockSpec((blk, D_EDGE), lambda i: (i, 0)),
            pl.BlockSpec((blk, D_EDGE), lambda i: (i, 0)),
            pl.BlockSpec((blk, D_NODE), lambda i: (i, 0)),
        ],
        out_shape=[
            jax.ShapeDtypeStruct((N_NODES, D_EDGE), jnp.float32),
            jax.ShapeDtypeStruct((N_NODES, D_EDGE), jnp.float32),
            jax.ShapeDtypeStruct((N_NODES, D_NODE), jnp.float32),
        ],
    )(node_attr, We_r, We_c, Wn_n)


PACK = 128 // D_EDGE
N_EP = N_EDGES // PACK


def _tc_edges_body(ea_ref, wbd_ref, g_ref, wg_ref, be_ref, e_ref):
    c16 = (jnp.dot(g_ref[...], wg_ref[...], preferred_element_type=jnp.float32)
           + be_ref[...])
    c128 = jnp.concatenate([c16] * PACK, axis=1)
    e_ref[...] = (jnp.dot(ea_ref[...], wbd_ref[...],
                          preferred_element_type=jnp.float32) + c128)


def _tc_edges(ea_packed, W_bd, global_attr, We_g, b_edge):
    blk = 4000
    grid = (N_EP // blk,)
    return pl.pallas_call(
        _tc_edges_body,
        grid=grid,
        in_specs=[
            pl.BlockSpec((blk, 128), lambda i: (i, 0)),
            pl.BlockSpec((128, 128), lambda i: (0, 0)),
            pl.BlockSpec((1, D_GLOBAL), lambda i: (0, 0)),
            pl.BlockSpec((D_GLOBAL, D_EDGE), lambda i: (0, 0)),
            pl.BlockSpec((1, D_EDGE), lambda i: (0, 0)),
        ],
        out_specs=pl.BlockSpec((blk, 128), lambda i: (i, 0)),
        out_shape=jax.ShapeDtypeStruct((N_EP, 128), jnp.float32),
    )(ea_packed, W_bd, global_attr, We_g, b_edge)


PC = C // PACK


def _sc_edge_body(A_hbm, B_hbm, E_hbm, ei_hbm,
                  ne_hbm, agg_hbm,
                  row_v, col_v, a_v, b_v, e_v, o_s, z_v, agg_sh,
                  sem_a0, sem_a1, sem_b0, sem_b1, sem_e0, sem_e1,
                  sem_st0, sem_st1):
    sem_a = (sem_a0, sem_a1)
    sem_b = (sem_b0, sem_b1)
    sem_e = (sem_e0, sem_e1)
    sem_st = (sem_st0, sem_st1)
    cid = lax.axis_index("c")
    sid = lax.axis_index("s")
    wid = sid * NC + cid
    base = wid * EPT
    base_p = wid * (EPT // PACK)

    def zfill(i, _):
        z_v[i, :] = jnp.zeros((16,), jnp.float32)
        return 0
    lax.fori_loop(0, ZROWS, zfill, 0, unroll=8)
    pltpu.sync_copy(z_v, agg_sh.at[pl.ds(sid * ZROWS, ZROWS)])

    pltpu.sync_copy(ei_hbm.at[0, pl.ds(base, EPT)], row_v)
    pltpu.sync_copy(ei_hbm.at[1, pl.ds(base, EPT)], col_v)

    def issue(j, b):
        pltpu.async_copy(A_hbm.at[row_v.at[pl.ds(j * C, C)]], a_v.at[b],
                         sem_a[b])
        pltpu.async_copy(B_hbm.at[col_v.at[pl.ds(j * C, C)]], b_v.at[b],
                         sem_b[b])
        pltpu.async_copy(E_hbm.at[pl.ds(base_p + j * PC, PC)], e_v.at[b],
                         sem_e[b])

    def wait_loads(j, b):
        pltpu.make_async_copy(A_hbm.at[row_v.at[pl.ds(j * C, C)]], a_v.at[b],
                              sem_a[b]).wait()
        pltpu.make_async_copy(B_hbm.at[col_v.at[pl.ds(j * C, C)]], b_v.at[b],
                              sem_b[b]).wait()
        pltpu.make_async_copy(E_hbm.at[pl.ds(base_p + j * PC, PC)], e_v.at[b],
                              sem_e[b]).wait()

    def compute_store(j, b):
        @pl.when(j >= 2)
        def _():
            pltpu.make_async_copy(o_s.at[b],
                                  ne_hbm.at[pl.ds(base + (j - 2) * C, C)],
                                  sem_st[b]).wait()
        wait_loads(j, b)

        def rowfn(q, _):
            for l in range(PACK):
                i = q * PACK + l
                o_s[b, i, :] = jnp.maximum(
                    a_v[b, i, :] + b_v[b, i, :]
                    + e_v[b, q, l * D_EDGE:(l + 1) * D_EDGE], 0.0)
            return 0
        lax.fori_loop(0, PC, rowfn, 0)

        pltpu.async_copy(o_s.at[b], ne_hbm.at[pl.ds(base + j * C, C)],
                         sem_st[b])
        pltpu.sync_copy(o_s.at[b], agg_sh.at[col_v.at[pl.ds(j * C, C)]],
                        add=True)

    issue(0, 0)
    plsc.subcore_barrier()

    def pair(p, _):
        j0 = p * 2

        @pl.when(j0 + 1 < CHUNKS)
        def _():
            issue(j0 + 1, 1)
        compute_store(j0, 0)

        @pl.when(j0 + 2 < CHUNKS)
        def _():
            issue(j0 + 2, 0)

        @pl.when(j0 + 1 < CHUNKS)
        def _():
            compute_store(j0 + 1, 1)
        return 0
    lax.fori_loop(0, (CHUNKS + 1) // 2, pair, 0)

    for b in (0, 1):
        j_last = CHUNKS - 1 if (CHUNKS - 1) % 2 == b else CHUNKS - 2
        pltpu.make_async_copy(o_s.at[b],
                              ne_hbm.at[pl.ds(base + j_last * C, C)],
                              sem_st[b]).wait()

    plsc.subcore_barrier()
    pltpu.sync_copy(agg_sh.at[pl.ds(sid * ZROWS, ZROWS)],
                    agg_hbm.at[cid, pl.ds(sid * ZROWS, ZROWS)])


def _sc_edge(A, B, E_packed, edge_index):
    mesh = plsc.VectorSubcoreMesh(core_axis_name="c", subcore_axis_name="s",
                                  num_cores=NC, num_subcores=NS)
    return pl.kernel(
        _sc_edge_body,
        out_type=(
            jax.ShapeDtypeStruct((N_EDGES, D_EDGE), jnp.float32),
            jax.ShapeDtypeStruct((NC, N_AGG, D_EDGE), jnp.float32),
        ),
        mesh=mesh,
        compiler_params=pltpu.CompilerParams(use_tc_tiling_on_sc=False),
        scratch_types=[
            pltpu.VMEM((EPT,), jnp.int32),
            pltpu.VMEM((EPT,), jnp.int32),
            pltpu.VMEM((2, C, D_EDGE), jnp.float32),
            pltpu.VMEM((2, C, D_EDGE), jnp.float32),
            pltpu.VMEM((2, PC, 128), jnp.float32),
            pltpu.VMEM((2, C, D_EDGE), jnp.float32),
            pltpu.VMEM((ZROWS, D_EDGE), jnp.float32),
            pltpu.VMEM_SHARED((N_AGG, D_EDGE), jnp.float32),
            pltpu.SemaphoreType.DMA,
            pltpu.SemaphoreType.DMA,
            pltpu.SemaphoreType.DMA,
            pltpu.SemaphoreType.DMA,
            pltpu.SemaphoreType.DMA,
            pltpu.SemaphoreType.DMA,
            pltpu.SemaphoreType.DMA,
            pltpu.SemaphoreType.DMA,
        ],
    )(A, B, E_packed, edge_index)


def _tc_node_global_body(agg_ref, p_ref, wna_ref, bn_ref,
                         wgn_ref, wge_ref, wgg_ref, bg_ref, g_ref,
                         nn_ref, gout_ref, accn, acce):
    i = pl.program_id(0)
    aggb = agg_ref[0] + agg_ref[1]
    nn = jnp.maximum(
        p_ref[...]
        + jnp.dot(aggb, wna_ref[...], preferred_element_type=jnp.float32)
        + bn_ref[...], 0.0)
    nn_ref[...] = nn

    @pl.when(i == 0)
    def _():
        accn[...] = jnp.zeros_like(accn)
        acce[...] = jnp.zeros_like(acce)

    accn[...] += jnp.sum(nn, axis=0, keepdims=True)
    acce[...] += jnp.sum(aggb, axis=0, keepdims=True)

    @pl.when(i == pl.num_programs(0) - 1)
    def _():
        nm = accn[...] / float(N_NODES)
        em = acce[...] / float(N_EDGES)
        gout_ref[...] = (
            jnp.dot(nm, wgn_ref[...], preferred_element_type=jnp.float32)
            + jnp.dot(em, wge_ref[...], preferred_element_type=jnp.float32)
            + jnp.dot(g_ref[...], wgg_ref[...], preferred_element_type=jnp.float32)
            + bg_ref[...])


def _tc_node_global(agg_parts, P, Wn_a, b_node, Wg_n, Wg_e, Wg_g,
                    b_global, global_attr):
    blk = 1000
    grid = (N_NODES // blk,)
    return pl.pallas_call(
        _tc_node_global_body,
        grid=grid,
        in_specs=[
            pl.BlockSpec((NC, blk, D_EDGE), lambda i: (0, i, 0)),
            pl.BlockSpec((blk, D_NODE), lambda i: (i, 0)),
            pl.BlockSpec((D_EDGE, D_NODE), lambda i: (0, 0)),
            pl.BlockSpec((1, D_NODE), lambda i: (0, 0)),
            pl.BlockSpec((D_NODE, D_GLOBAL), lambda i: (0, 0)),
            pl.BlockSpec((D_EDGE, D_GLOBAL), lambda i: (0, 0)),
            pl.BlockSpec((D_GLOBAL, D_GLOBAL), lambda i: (0, 0)),
            pl.BlockSpec((1, D_GLOBAL), lambda i: (0, 0)),
            pl.BlockSpec((1, D_GLOBAL), lambda i: (0, 0)),
        ],
        out_specs=[
            pl.BlockSpec((blk, D_NODE), lambda i: (i, 0)),
            pl.BlockSpec((1, D_GLOBAL), lambda i: (0, 0)),
        ],
        out_shape=[
            jax.ShapeDtypeStruct((N_NODES, D_NODE), jnp.float32),
            jax.ShapeDtypeStruct((1, D_GLOBAL), jnp.float32),
        ],
        scratch_shapes=[
            pltpu.VMEM((1, D_NODE), jnp.float32),
            pltpu.VMEM((1, D_EDGE), jnp.float32),
        ],
    )(agg_parts, P, Wn_a, b_node, Wg_n, Wg_e, Wg_g, b_global, global_attr)


def kernel(node_attr, edge_index, edge_attr, global_attr,
           W_edge, b_edge, W_node, b_node, W_global, b_global):
    We_r = W_edge[:D_NODE]
    We_c = W_edge[D_NODE:2 * D_NODE]
    We_e = W_edge[2 * D_NODE:2 * D_NODE + D_EDGE]
    We_g = W_edge[2 * D_NODE + D_EDGE:]
    Wn_n = W_node[:D_NODE]
    Wn_a = W_node[D_NODE:]
    Wg_n = W_global[:D_NODE]
    Wg_e = W_global[D_NODE:D_NODE + D_EDGE]
    Wg_g = W_global[D_NODE + D_EDGE:]

    A, B, P = _tc_nodes(node_attr, We_r, We_c, Wn_n)
    ea_packed = edge_attr.reshape(N_EP, 128)
    W_bd = jnp.kron(jnp.eye(PACK, dtype=jnp.float32), We_e)
    E_packed = _tc_edges(ea_packed, W_bd, global_attr, We_g,
                         b_edge.reshape(1, D_EDGE))

    ei = edge_index.astype(jnp.int32)
    new_edge, agg_parts = _sc_edge(A, B, E_packed, ei)

    new_node, new_global = _tc_node_global(
        agg_parts, P, Wn_a, b_node.reshape(1, D_NODE),
        Wg_n, Wg_e, Wg_g, b_global.reshape(1, D_GLOBAL), global_attr)

    return new_node, new_edge, new_global


# speedup vs baseline: 1.0703x; 1.0703x over previous
import functools

import jax
import jax.numpy as jnp
from jax import lax
from jax.experimental import pallas as pl
from jax.experimental.pallas import tpu as pltpu
from jax.experimental.pallas import tpu_sc as plsc

N_NODES = 10000
N_EDGES = 320000
D_NODE = 128
D_EDGE = 16
D_GLOBAL = 32

NC = 2
NS = 16
NW = NC * NS
EPT = N_EDGES // NW
C = 200
CHUNKS = EPT // C
N_AGG = 10240
ZROWS = N_AGG // NS


def _tc_nodes_body(x_ref, wr_ref, wc_ref, wn_ref, a_ref, b_ref, p_ref):
    x = x_ref[...]
    a_ref[...] = jnp.dot(x, wr_ref[...], preferred_element_type=jnp.float32)
    b_ref[...] = jnp.dot(x, wc_ref[...], preferred_element_type=jnp.float32)
    p_ref[...] = jnp.dot(x, wn_ref[...], preferred_element_type=jnp.float32)


def _tc_nodes(node_attr, We_r, We_c, Wn_n):
    blk = 1000
    grid = (N_NODES // blk,)
    return pl.pallas_call(
        _tc_nodes_body,
        grid=grid,
        in_specs=[
            pl.BlockSpec((blk, D_NODE), lambda i: (i, 0)),
            pl.BlockSpec((D_NODE, D_EDGE), lambda i: (0, 0)),
            pl.BlockSpec((D_NODE, D_EDGE), lambda i: (0, 0)),
            pl.BlockSpec((D_NODE, D_NODE), lambda i: (0, 0)),
        ],
        out_specs=[
            pl.BlockSpec((blk, D_EDGE), lambda i: (i, 0)),
            pl.BlockSpec((blk, D_EDGE), lambda i: (i, 0)),
            pl.BlockSpec((blk, D_NODE), lambda i: (i, 0)),
        ],
        out_shape=[
            jax.ShapeDtypeStruct((N_NODES, D_EDGE), jnp.float32),
            jax.ShapeDtypeStruct((N_NODES, D_EDGE), jnp.float32),
            jax.ShapeDtypeStruct((N_NODES, D_NODE), jnp.float32),
        ],
    )(node_attr, We_r, We_c, Wn_n)


PACK = 128 // D_EDGE
N_EP = N_EDGES // PACK


def _tc_edges_body(ea_ref, wbd_ref, g_ref, wg_ref, be_ref, e_ref):
    c16 = (jnp.dot(g_ref[...], wg_ref[...], preferred_element_type=jnp.float32)
           + be_ref[...])
    c128 = jnp.concatenate([c16] * PACK, axis=1)
    e_ref[...] = (jnp.dot(ea_ref[...], wbd_ref[...],
                          preferred_element_type=jnp.float32) + c128)


def _tc_edges(ea_packed, W_bd, global_attr, We_g, b_edge):
    blk = 4000
    grid = (N_EP // blk,)
    return pl.pallas_call(
        _tc_edges_body,
        grid=grid,
        in_specs=[
            pl.BlockSpec((blk, 128), lambda i: (i, 0)),
            pl.BlockSpec((128, 128), lambda i: (0, 0)),
            pl.BlockSpec((1, D_GLOBAL), lambda i: (0, 0)),
            pl.BlockSpec((D_GLOBAL, D_EDGE), lambda i: (0, 0)),
            pl.BlockSpec((1, D_EDGE), lambda i: (0, 0)),
        ],
        out_specs=pl.BlockSpec((blk, 128), lambda i: (i, 0)),
        out_shape=jax.ShapeDtypeStruct((N_EP, 128), jnp.float32),
    )(ea_packed, W_bd, global_attr, We_g, b_edge)


PC = C // PACK


def _sc_edge_body(A_hbm, B_hbm, E_hbm, ei_hbm,
                  ne_hbm, agg_hbm,
                  row_v, col_v, a_v, b_v, e_v, o_s, z_v, agg_sh,
                  sem_a0, sem_a1, sem_b0, sem_b1, sem_e0, sem_e1,
                  sem_st0, sem_st1):
    sem_a = (sem_a0, sem_a1)
    sem_b = (sem_b0, sem_b1)
    sem_e = (sem_e0, sem_e1)
    sem_st = (sem_st0, sem_st1)
    cid = lax.axis_index("c")
    sid = lax.axis_index("s")
    wid = sid * NC + cid
    base = wid * EPT
    base_p = wid * (EPT // PACK)

    def zfill(i, _):
        z_v[i, :] = jnp.zeros((16,), jnp.float32)
        return 0
    lax.fori_loop(0, ZROWS, zfill, 0, unroll=8)
    pltpu.sync_copy(z_v, agg_sh.at[pl.ds(sid * ZROWS, ZROWS)])

    pltpu.sync_copy(ei_hbm.at[0, pl.ds(base, EPT)], row_v)
    pltpu.sync_copy(ei_hbm.at[1, pl.ds(base, EPT)], col_v)

    def issue(j, b):
        pltpu.async_copy(A_hbm.at[row_v.at[pl.ds(j * C, C)]], a_v.at[b],
                         sem_a[b])
        pltpu.async_copy(B_hbm.at[col_v.at[pl.ds(j * C, C)]], b_v.at[b],
                         sem_b[b])
        pltpu.async_copy(E_hbm.at[pl.ds(base_p + j * PC, PC)], e_v.at[b],
                         sem_e[b])

    def wait_loads(j, b):
        pltpu.make_async_copy(A_hbm.at[row_v.at[pl.ds(j * C, C)]], a_v.at[b],
                              sem_a[b]).wait()
        pltpu.make_async_copy(B_hbm.at[col_v.at[pl.ds(j * C, C)]], b_v.at[b],
                              sem_b[b]).wait()
        pltpu.make_async_copy(E_hbm.at[pl.ds(base_p + j * PC, PC)], e_v.at[b],
                              sem_e[b]).wait()

    def compute_store(j, b):
        @pl.when(j >= 2)
        def _():
            pltpu.make_async_copy(o_s.at[b],
                                  ne_hbm.at[pl.ds(base + (j - 2) * C, C)],
                                  sem_st[b]).wait()
        wait_loads(j, b)

        def rowfn(q, _):
            for l in range(PACK):
                i = q * PACK + l
                o_s[b, i, :] = jnp.maximum(
                    a_v[b, i, :] + b_v[b, i, :]
                    + e_v[b, q, l * D_EDGE:(l + 1) * D_EDGE], 0.0)
            return 0
        lax.fori_loop(0, PC, rowfn, 0)

        pltpu.async_copy(o_s.at[b], ne_hbm.at[pl.ds(base + j * C, C)],
                         sem_st[b])
        pltpu.sync_copy(o_s.at[b], agg_sh.at[col_v.at[pl.ds(j * C, C)]],
                        add=True)

    issue(0, 0)
    plsc.subcore_barrier()

    def pair(p, _):
        j0 = p * 2

        @pl.when(j0 + 1 < CHUNKS)
        def _():
            issue(j0 + 1, 1)
        compute_store(j0, 0)

        @pl.when(j0 + 2 < CHUNKS)
        def _():
            issue(j0 + 2, 0)

        @pl.when(j0 + 1 < CHUNKS)
        def _():
            compute_store(j0 + 1, 1)
        return 0
    lax.fori_loop(0, (CHUNKS + 1) // 2, pair, 0)

    for b in (0, 1):
        j_last = CHUNKS - 1 if (CHUNKS - 1) % 2 == b else CHUNKS - 2
        pltpu.make_async_copy(o_s.at[b],
                              ne_hbm.at[pl.ds(base + j_last * C, C)],
                              sem_st[b]).wait()

    plsc.subcore_barrier()
    pltpu.sync_copy(agg_sh.at[pl.ds(sid * ZROWS, ZROWS)],
                    agg_hbm.at[cid, pl.ds(sid * ZROWS, ZROWS)])


def _sc_edge(A, B, E_packed, edge_index):
    mesh = plsc.VectorSubcoreMesh(core_axis_name="c", subcore_axis_name="s",
                                  num_cores=NC, num_subcores=NS)
    return pl.kernel(
        _sc_edge_body,
        out_type=(
            jax.ShapeDtypeStruct((N_EDGES, D_EDGE), jnp.float32),
            jax.ShapeDtypeStruct((NC, N_AGG, D_EDGE), jnp.float32),
        ),
        mesh=mesh,
        compiler_params=pltpu.CompilerParams(use_tc_tiling_on_sc=False),
        scratch_types=[
            pltpu.VMEM((EPT,), jnp.int32),
            pltpu.VMEM((EPT,), jnp.int32),
            pltpu.VMEM((2, C, D_EDGE), jnp.float32),
            pltpu.VMEM((2, C, D_EDGE), jnp.float32),
            pltpu.VMEM((2, PC, 128), jnp.float32),
            pltpu.VMEM((2, C, D_EDGE), jnp.float32),
            pltpu.VMEM((ZROWS, D_EDGE), jnp.float32),
            pltpu.VMEM_SHARED((N_AGG, D_EDGE), jnp.float32),
            pltpu.SemaphoreType.DMA,
            pltpu.SemaphoreType.DMA,
            pltpu.SemaphoreType.DMA,
            pltpu.SemaphoreType.DMA,
            pltpu.SemaphoreType.DMA,
            pltpu.SemaphoreType.DMA,
            pltpu.SemaphoreType.DMA,
            pltpu.SemaphoreType.DMA,
        ],
    )(A, B, E_packed, edge_index)


def _tc_node_global_body(agg_ref, p_ref, wna_ref, bn_ref,
                         wgn_ref, wge_ref, wgg_ref, bg_ref, g_ref,
                         nn_ref, gout_ref, accn, acce):
    i = pl.program_id(0)
    aggb = agg_ref[0] + agg_ref[1]
    nn = jnp.maximum(
        p_ref[...]
        + jnp.dot(aggb, wna_ref[...], preferred_element_type=jnp.float32)
        + bn_ref[...], 0.0)
    nn_ref[...] = nn

    @pl.when(i == 0)
    def _():
        accn[...] = jnp.zeros_like(accn)
        acce[...] = jnp.zeros_like(acce)

    accn[...] += jnp.sum(nn, axis=0, keepdims=True)
    acce[...] += jnp.sum(aggb, axis=0, keepdims=True)

    @pl.when(i == pl.num_programs(0) - 1)
    def _():
        nm = accn[...] / float(N_NODES)
        em = acce[...] / float(N_EDGES)
        gout_ref[...] = (
            jnp.dot(nm, wgn_ref[...], preferred_element_type=jnp.float32)
            + jnp.dot(em, wge_ref[...], preferred_element_type=jnp.float32)
            + jnp.dot(g_ref[...], wgg_ref[...], preferred_element_type=jnp.float32)
            + bg_ref[...])


def _tc_node_global(agg_parts, P, Wn_a, b_node, Wg_n, Wg_e, Wg_g,
                    b_global, global_attr):
    blk = 1000
    grid = (N_NODES // blk,)
    return pl.pallas_call(
        _tc_node_global_body,
        grid=grid,
        in_specs=[
            pl.BlockSpec((NC, blk, D_EDGE), lambda i: (0, i, 0)),
            pl.BlockSpec((blk, D_NODE), lambda i: (i, 0)),
            pl.BlockSpec((D_EDGE, D_NODE), lambda i: (0, 0)),
            pl.BlockSpec((1, D_NODE), lambda i: (0, 0)),
            pl.BlockSpec((D_NODE, D_GLOBAL), lambda i: (0, 0)),
            pl.BlockSpec((D_EDGE, D_GLOBAL), lambda i: (0, 0)),
            pl.BlockSpec((D_GLOBAL, D_GLOBAL), lambda i: (0, 0)),
            pl.BlockSpec((1, D_GLOBAL), lambda i: (0, 0)),
            pl.BlockSpec((1, D_GLOBAL), lambda i: (0, 0)),
        ],
        out_specs=[
            pl.BlockSpec((blk, D_NODE), lambda i: (i, 0)),
            pl.BlockSpec((1, D_GLOBAL), lambda i: (0, 0)),
        ],
        out_shape=[
            jax.ShapeDtypeStruct((N_NODES, D_NODE), jnp.float32),
            jax.ShapeDtypeStruct((1, D_GLOBAL), jnp.float32),
        ],
        scratch_shapes=[
            pltpu.VMEM((1, D_NODE), jnp.float32),
            pltpu.VMEM((1, D_EDGE), jnp.float32),
        ],
    )(agg_parts, P, Wn_a, b_node, Wg_n, Wg_e, Wg_g, b_global, global_attr)


def kernel(node_attr, edge_index, edge_attr, global_attr,
           W_edge, b_edge, W_node, b_node, W_global, b_global):
    We_r = W_edge[:D_NODE]
    We_c = W_edge[D_NODE:2 * D_NODE]
    We_e = W_edge[2 * D_NODE:2 * D_NODE + D_EDGE]
    We_g = W_edge[2 * D_NODE + D_EDGE:]
    Wn_n = W_node[:D_NODE]
    Wn_a = W_node[D_NODE:]
    Wg_n = W_global[:D_NODE]
    Wg_e = W_global[D_NODE:D_NODE + D_EDGE]
    Wg_g = W_global[D_NODE + D_EDGE:]

    A, B, P = _tc_nodes(node_attr, We_r, We_c, Wn_n)
    ea_packed = edge_attr.reshape(N_EP, 128)
    W_bd = jnp.kron(jnp.eye(PACK, dtype=jnp.float32), We_e)
    E_packed = _tc_edges(ea_packed, W_bd, global_attr, We_g,
                         b_edge.reshape(1, D_EDGE))

    ei = edge_index.astype(jnp.int32)
    new_edge, agg_parts = _sc_edge(A, B, E_packed, ei)

    new_node, new_global = _tc_node_global(
        agg_parts, P, Wn_a, b_node.reshape(1, D_NODE),
        Wg_n, Wg_e, Wg_g, b_global.reshape(1, D_GLOBAL), global_attr)

    return new_node, new_edge, new_global


# speedup vs baseline: 1.0997x; 1.0275x over previous
import functools

import jax
import jax.numpy as jnp
from jax import lax
from jax.experimental import pallas as pl
from jax.experimental.pallas import tpu as pltpu
from jax.experimental.pallas import tpu_sc as plsc

N_NODES = 10000
N_EDGES = 320000
D_NODE = 128
D_EDGE = 16
D_GLOBAL = 32

NC = 2
NS = 16
NW = NC * NS
EPT = N_EDGES // NW
C = 400
CHUNKS = EPT // C
N_AGG = 10240
ZROWS = N_AGG // NS


def _tc_nodes_body(x_ref, wr_ref, wc_ref, wn_ref, a_ref, b_ref, p_ref):
    x = x_ref[...]
    a_ref[...] = jnp.dot(x, wr_ref[...], preferred_element_type=jnp.float32)
    b_ref[...] = jnp.dot(x, wc_ref[...], preferred_element_type=jnp.float32)
    p_ref[...] = jnp.dot(x, wn_ref[...], preferred_element_type=jnp.float32)


def _tc_nodes(node_attr, We_r, We_c, Wn_n):
    blk = 1000
    grid = (N_NODES // blk,)
    return pl.pallas_call(
        _tc_nodes_body,
        grid=grid,
        in_specs=[
            pl.BlockSpec((blk, D_NODE), lambda i: (i, 0)),
            pl.BlockSpec((D_NODE, D_EDGE), lambda i: (0, 0)),
            pl.BlockSpec((D_NODE, D_EDGE), lambda i: (0, 0)),
            pl.BlockSpec((D_NODE, D_NODE), lambda i: (0, 0)),
        ],
        out_specs=[
            pl.BlockSpec((blk, D_EDGE), lambda i: (i, 0)),
            pl.BlockSpec((blk, D_EDGE), lambda i: (i, 0)),
            pl.BlockSpec((blk, D_NODE), lambda i: (i, 0)),
        ],
        out_shape=[
            jax.ShapeDtypeStruct((N_NODES, D_EDGE), jnp.float32),
            jax.ShapeDtypeStruct((N_NODES, D_EDGE), jnp.float32),
            jax.ShapeDtypeStruct((N_NODES, D_NODE), jnp.float32),
        ],
    )(node_attr, We_r, We_c, Wn_n)


PACK = 128 // D_EDGE
N_EP = N_EDGES // PACK


def _tc_edges_body(ea_ref, wbd_ref, g_ref, wg_ref, be_ref, e_ref):
    c16 = (jnp.dot(g_ref[...], wg_ref[...], preferred_element_type=jnp.float32)
           + be_ref[...])
    c128 = jnp.concatenate([c16] * PACK, axis=1)
    e_ref[...] = (jnp.dot(ea_ref[...], wbd_ref[...],
                          preferred_element_type=jnp.float32) + c128)


def _tc_edges(ea_packed, W_bd, global_attr, We_g, b_edge):
    blk = 4000
    grid = (N_EP // blk,)
    return pl.pallas_call(
        _tc_edges_body,
        grid=grid,
        in_specs=[
            pl.BlockSpec((blk, 128), lambda i: (i, 0)),
            pl.BlockSpec((128, 128), lambda i: (0, 0)),
            pl.BlockSpec((1, D_GLOBAL), lambda i: (0, 0)),
            pl.BlockSpec((D_GLOBAL, D_EDGE), lambda i: (0, 0)),
            pl.BlockSpec((1, D_EDGE), lambda i: (0, 0)),
        ],
        out_specs=pl.BlockSpec((blk, 128), lambda i: (i, 0)),
        out_shape=jax.ShapeDtypeStruct((N_EP, 128), jnp.float32),
    )(ea_packed, W_bd, global_attr, We_g, b_edge)


PC = C // PACK


def _sc_edge_body(A_hbm, B_hbm, E_hbm, ei_hbm,
                  ne_hbm, agg_hbm,
                  row_v, col_v, a_v, b_v, e_v, o_s, z_v, agg_sh,
                  sem_a0, sem_a1, sem_b0, sem_b1, sem_e0, sem_e1,
                  sem_st0, sem_st1):
    sem_a = (sem_a0, sem_a1)
    sem_b = (sem_b0, sem_b1)
    sem_e = (sem_e0, sem_e1)
    sem_st = (sem_st0, sem_st1)
    cid = lax.axis_index("c")
    sid = lax.axis_index("s")
    wid = sid * NC + cid
    base = wid * EPT
    base_p = wid * (EPT // PACK)

    def zfill(i, _):
        z_v[i, :] = jnp.zeros((16,), jnp.float32)
        return 0
    lax.fori_loop(0, ZROWS, zfill, 0, unroll=8)
    pltpu.sync_copy(z_v, agg_sh.at[pl.ds(sid * ZROWS, ZROWS)])

    pltpu.sync_copy(ei_hbm.at[0, pl.ds(base, EPT)], row_v)
    pltpu.sync_copy(ei_hbm.at[1, pl.ds(base, EPT)], col_v)

    def issue(j, b):
        pltpu.async_copy(A_hbm.at[row_v.at[pl.ds(j * C, C)]], a_v.at[b],
                         sem_a[b])
        pltpu.async_copy(B_hbm.at[col_v.at[pl.ds(j * C, C)]], b_v.at[b],
                         sem_b[b])
        pltpu.async_copy(E_hbm.at[pl.ds(base_p + j * PC, PC)], e_v.at[b],
                         sem_e[b])

    def wait_loads(j, b):
        pltpu.make_async_copy(A_hbm.at[row_v.at[pl.ds(j * C, C)]], a_v.at[b],
                              sem_a[b]).wait()
        pltpu.make_async_copy(B_hbm.at[col_v.at[pl.ds(j * C, C)]], b_v.at[b],
                              sem_b[b]).wait()
        pltpu.make_async_copy(E_hbm.at[pl.ds(base_p + j * PC, PC)], e_v.at[b],
                              sem_e[b]).wait()

    def compute_store(j, b):
        @pl.when(j >= 2)
        def _():
            pltpu.make_async_copy(o_s.at[b],
                                  ne_hbm.at[pl.ds(base + (j - 2) * C, C)],
                                  sem_st[b]).wait()
        wait_loads(j, b)

        def rowfn(q, _):
            for l in range(PACK):
                i = q * PACK + l
                o_s[b, i, :] = jnp.maximum(
                    a_v[b, i, :] + b_v[b, i, :]
                    + e_v[b, q, l * D_EDGE:(l + 1) * D_EDGE], 0.0)
            return 0
        lax.fori_loop(0, PC, rowfn, 0)

        pltpu.async_copy(o_s.at[b], ne_hbm.at[pl.ds(base + j * C, C)],
                         sem_st[b])
        pltpu.sync_copy(o_s.at[b], agg_sh.at[col_v.at[pl.ds(j * C, C)]],
                        add=True)

    issue(0, 0)
    plsc.subcore_barrier()

    def pair(p, _):
        j0 = p * 2

        @pl.when(j0 + 1 < CHUNKS)
        def _():
            issue(j0 + 1, 1)
        compute_store(j0, 0)

        @pl.when(j0 + 2 < CHUNKS)
        def _():
            issue(j0 + 2, 0)

        @pl.when(j0 + 1 < CHUNKS)
        def _():
            compute_store(j0 + 1, 1)
        return 0
    lax.fori_loop(0, (CHUNKS + 1) // 2, pair, 0)

    for b in (0, 1):
        j_last = CHUNKS - 1 if (CHUNKS - 1) % 2 == b else CHUNKS - 2
        pltpu.make_async_copy(o_s.at[b],
                              ne_hbm.at[pl.ds(base + j_last * C, C)],
                              sem_st[b]).wait()

    plsc.subcore_barrier()
    pltpu.sync_copy(agg_sh.at[pl.ds(sid * ZROWS, ZROWS)],
                    agg_hbm.at[cid, pl.ds(sid * ZROWS, ZROWS)])


def _sc_edge(A, B, E_packed, edge_index):
    mesh = plsc.VectorSubcoreMesh(core_axis_name="c", subcore_axis_name="s",
                                  num_cores=NC, num_subcores=NS)
    return pl.kernel(
        _sc_edge_body,
        out_type=(
            jax.ShapeDtypeStruct((N_EDGES, D_EDGE), jnp.float32),
            jax.ShapeDtypeStruct((NC, N_AGG, D_EDGE), jnp.float32),
        ),
        mesh=mesh,
        compiler_params=pltpu.CompilerParams(use_tc_tiling_on_sc=False),
        scratch_types=[
            pltpu.VMEM((EPT,), jnp.int32),
            pltpu.VMEM((EPT,), jnp.int32),
            pltpu.VMEM((2, C, D_EDGE), jnp.float32),
            pltpu.VMEM((2, C, D_EDGE), jnp.float32),
            pltpu.VMEM((2, PC, 128), jnp.float32),
            pltpu.VMEM((2, C, D_EDGE), jnp.float32),
            pltpu.VMEM((ZROWS, D_EDGE), jnp.float32),
            pltpu.VMEM_SHARED((N_AGG, D_EDGE), jnp.float32),
            pltpu.SemaphoreType.DMA,
            pltpu.SemaphoreType.DMA,
            pltpu.SemaphoreType.DMA,
            pltpu.SemaphoreType.DMA,
            pltpu.SemaphoreType.DMA,
            pltpu.SemaphoreType.DMA,
            pltpu.SemaphoreType.DMA,
            pltpu.SemaphoreType.DMA,
        ],
    )(A, B, E_packed, edge_index)


def _tc_node_global_body(agg_ref, p_ref, wna_ref, bn_ref,
                         wgn_ref, wge_ref, wgg_ref, bg_ref, g_ref,
                         nn_ref, gout_ref, accn, acce):
    i = pl.program_id(0)
    aggb = agg_ref[0] + agg_ref[1]
    nn = jnp.maximum(
        p_ref[...]
        + jnp.dot(aggb, wna_ref[...], preferred_element_type=jnp.float32)
        + bn_ref[...], 0.0)
    nn_ref[...] = nn

    @pl.when(i == 0)
    def _():
        accn[...] = jnp.zeros_like(accn)
        acce[...] = jnp.zeros_like(acce)

    accn[...] += jnp.sum(nn, axis=0, keepdims=True)
    acce[...] += jnp.sum(aggb, axis=0, keepdims=True)

    @pl.when(i == pl.num_programs(0) - 1)
    def _():
        nm = accn[...] / float(N_NODES)
        em = acce[...] / float(N_EDGES)
        gout_ref[...] = (
            jnp.dot(nm, wgn_ref[...], preferred_element_type=jnp.float32)
            + jnp.dot(em, wge_ref[...], preferred_element_type=jnp.float32)
            + jnp.dot(g_ref[...], wgg_ref[...], preferred_element_type=jnp.float32)
            + bg_ref[...])


def _tc_node_global(agg_parts, P, Wn_a, b_node, Wg_n, Wg_e, Wg_g,
                    b_global, global_attr):
    blk = 1000
    grid = (N_NODES // blk,)
    return pl.pallas_call(
        _tc_node_global_body,
        grid=grid,
        in_specs=[
            pl.BlockSpec((NC, blk, D_EDGE), lambda i: (0, i, 0)),
            pl.BlockSpec((blk, D_NODE), lambda i: (i, 0)),
            pl.BlockSpec((D_EDGE, D_NODE), lambda i: (0, 0)),
            pl.BlockSpec((1, D_NODE), lambda i: (0, 0)),
            pl.BlockSpec((D_NODE, D_GLOBAL), lambda i: (0, 0)),
            pl.BlockSpec((D_EDGE, D_GLOBAL), lambda i: (0, 0)),
            pl.BlockSpec((D_GLOBAL, D_GLOBAL), lambda i: (0, 0)),
            pl.BlockSpec((1, D_GLOBAL), lambda i: (0, 0)),
            pl.BlockSpec((1, D_GLOBAL), lambda i: (0, 0)),
        ],
        out_specs=[
            pl.BlockSpec((blk, D_NODE), lambda i: (i, 0)),
            pl.BlockSpec((1, D_GLOBAL), lambda i: (0, 0)),
        ],
        out_shape=[
            jax.ShapeDtypeStruct((N_NODES, D_NODE), jnp.float32),
            jax.ShapeDtypeStruct((1, D_GLOBAL), jnp.float32),
        ],
        scratch_shapes=[
            pltpu.VMEM((1, D_NODE), jnp.float32),
            pltpu.VMEM((1, D_EDGE), jnp.float32),
        ],
    )(agg_parts, P, Wn_a, b_node, Wg_n, Wg_e, Wg_g, b_global, global_attr)


def kernel(node_attr, edge_index, edge_attr, global_attr,
           W_edge, b_edge, W_node, b_node, W_global, b_global):
    We_r = W_edge[:D_NODE]
    We_c = W_edge[D_NODE:2 * D_NODE]
    We_e = W_edge[2 * D_NODE:2 * D_NODE + D_EDGE]
    We_g = W_edge[2 * D_NODE + D_EDGE:]
    Wn_n = W_node[:D_NODE]
    Wn_a = W_node[D_NODE:]
    Wg_n = W_global[:D_NODE]
    Wg_e = W_global[D_NODE:D_NODE + D_EDGE]
    Wg_g = W_global[D_NODE + D_EDGE:]

    A, B, P = _tc_nodes(node_attr, We_r, We_c, Wn_n)
    ea_packed = edge_attr.reshape(N_EP, 128)
    W_bd = jnp.kron(jnp.eye(PACK, dtype=jnp.float32), We_e)
    E_packed = _tc_edges(ea_packed, W_bd, global_attr, We_g,
                         b_edge.reshape(1, D_EDGE))

    ei = edge_index.astype(jnp.int32)
    new_edge, agg_parts = _sc_edge(A, B, E_packed, ei)

    new_node, new_global = _tc_node_global(
        agg_parts, P, Wn_a, b_node.reshape(1, D_NODE),
        Wg_n, Wg_e, Wg_g, b_global.reshape(1, D_GLOBAL), global_attr)

    return new_node, new_edge, new_global
